# Initial kernel scaffold; baseline (speedup 1.0000x reference)
import jax
import jax.numpy as jnp
from jax.experimental import pallas as pl


def kernel(x, edge_index, edge_weights, W1, b1, W2, b2, W3, b3, Wl, bl):
    raise NotImplementedError("write your pallas kernel here")


# speedup vs baseline: 8.3699x; 8.3699x over previous
import functools

import jax
import jax.numpy as jnp
from jax import lax
from jax.experimental import pallas as pl
from jax.experimental.pallas import tpu as pltpu
from jax.experimental.pallas import tpu_sc as plsc

N = 10000
E = 320000
H = 128
C = 16

NC = 2
NS = 16
NW = NC * NS
EPT = E // NW
CH = 80
NCHUNK = EPT // CH
WB_CH = 624
WB_LAST = N - (NS - 1) * WB_CH

_MESH = plsc.VectorSubcoreMesh(core_axis_name="c", subcore_axis_name="s")
_SC_PARAMS = pltpu.CompilerParams(needs_layout_passes=False)


@functools.partial(
    pl.kernel,
    mesh=_MESH,
    out_type=jax.ShapeDtypeStruct((NC, N), jnp.float32),
    compiler_params=_SC_PARAMS,
    scratch_types=[
        pltpu.VMEM((CH,), jnp.int32),
        pltpu.VMEM((CH,), jnp.float32),
        pltpu.VMEM_SHARED((N,), jnp.float32),
    ],
)
def _sc_deg(dst_hbm, w_hbm, zero_hbm, out_hbm, didx, wv, acc):
    c = lax.axis_index("c")
    s = lax.axis_index("s")
    wid = s * NC + c
    base = wid * EPT

    @pl.when(s == 0)
    def _():
        pltpu.sync_copy(zero_hbm, acc)

    plsc.subcore_barrier()

    def body(j, carry):
        off = pl.multiple_of(base + j * CH, 8)
        pltpu.sync_copy(dst_hbm.at[pl.ds(off, CH)], didx)
        pltpu.sync_copy(w_hbm.at[pl.ds(off, CH)], wv)
        pltpu.sync_copy(wv, acc.at[didx], add=True)
        return carry

    lax.fori_loop(0, NCHUNK, body, 0)
    plsc.subcore_barrier()

    @pl.when(s == 0)
    def _():
        pltpu.sync_copy(acc, out_hbm.at[c])


@functools.partial(
    pl.kernel,
    mesh=_MESH,
    out_type=jax.ShapeDtypeStruct((NC, N, H), jnp.float32),
    compiler_params=_SC_PARAMS,
    scratch_types=[
        pltpu.VMEM((CH,), jnp.int32),
        pltpu.VMEM((CH,), jnp.int32),
        pltpu.VMEM((CH,), jnp.float32),
        pltpu.VMEM((CH, H), jnp.float32),
        pltpu.SemaphoreType.DMA,
        pltpu.VMEM_SHARED((N, H), jnp.float32),
    ],
)
def _sc_agg(hs_hbm, src_hbm, dst_hbm, w_hbm, zero_hbm, out_hbm,
            sidx, didx, wv, rows, sem, acc):
    c = lax.axis_index("c")
    s = lax.axis_index("s")
    wid = s * NC + c
    base = wid * EPT

    @pl.when(s == 0)
    def _():
        pltpu.sync_copy(zero_hbm, acc)

    plsc.subcore_barrier()

    def body(j, carry):
        off = pl.multiple_of(base + j * CH, 8)
        pltpu.sync_copy(src_hbm.at[pl.ds(off, CH)], sidx)
        pltpu.sync_copy(dst_hbm.at[pl.ds(off, CH)], didx)
        pltpu.sync_copy(w_hbm.at[pl.ds(off, CH)], wv)
        pltpu.async_copy(hs_hbm.at[sidx], rows, sem).wait()
        for g in range(CH // 16):
            wvec = wv[pl.ds(g * 16, 16)]
            for l in range(16):
                r = g * 16 + l
                wb = wvec.at[jnp.full((16,), l, jnp.int32)].get(
                    mode="promise_in_bounds")
                for col in range(H // 16):
                    sl = pl.ds(col * 16, 16)
                    rows[r, sl] = rows[r, sl] * wb
        pltpu.sync_copy(rows, acc.at[didx], add=True)
        return carry

    lax.fori_loop(0, NCHUNK, body, 0)
    plsc.subcore_barrier()

    @pl.when(s < NS - 1)
    def _():
        rsl = pl.ds(pl.multiple_of(s * WB_CH, 8), WB_CH)
        pltpu.sync_copy(acc.at[rsl], out_hbm.at[c].at[rsl])

    @pl.when(s == NS - 1)
    def _():
        rsl = pl.ds((NS - 1) * WB_CH, WB_LAST)
        pltpu.sync_copy(acc.at[rsl], out_hbm.at[c].at[rsl])


BR = 2000
_GRID = N // BR


def _dot(a, b):
    return jnp.dot(a, b, preferred_element_type=jnp.float32,
                   precision=lax.Precision.HIGHEST)


def _tc_pre(x, w1t, degs):
    def body(x_ref, w_ref, deg_ref, dinv_ref, hs_ref):
        deg = deg_ref[:, 0] + deg_ref[:, 1] + 1.0
        dinv = jnp.where(deg > 0, lax.rsqrt(deg), 0.0)
        dinv_ref[...] = dinv[:, None]
        hs_ref[...] = _dot(x_ref[...], w_ref[...]) * dinv[:, None]

    return pl.pallas_call(
        body,
        grid=(_GRID,),
        in_specs=[
            pl.BlockSpec((BR, H), lambda i: (i, 0)),
            pl.BlockSpec((H, H), lambda i: (0, 0)),
            pl.BlockSpec((BR, NC), lambda i: (i, 0)),
        ],
        out_specs=[
            pl.BlockSpec((BR, 1), lambda i: (i, 0)),
            pl.BlockSpec((BR, H), lambda i: (i, 0)),
        ],
        out_shape=[
            jax.ShapeDtypeStruct((N, 1), jnp.float32),
            jax.ShapeDtypeStruct((N, H), jnp.float32),
        ],
    )(x, w1t, degs.T)


def _layer_out(agg_ref, hs_ref, dinv_ref, b_ref):
    y = (agg_ref[0] + agg_ref[1] + hs_ref[...]) * dinv_ref[...] + b_ref[...]
    nrm = jnp.sqrt(jnp.sum(y * y, axis=1, keepdims=True))
    y = y / jnp.maximum(nrm, 1e-12)
    return jnp.maximum(y, 0.0)


def _tc_post(agg, hs, dinv, b, wnt):
    def body(agg_ref, hs_ref, dinv_ref, b_ref, w_ref, out_ref, hsn_ref):
        y = _layer_out(agg_ref, hs_ref, dinv_ref, b_ref)
        out_ref[...] = y
        hsn_ref[...] = _dot(y, w_ref[...]) * dinv_ref[...]

    return pl.pallas_call(
        body,
        grid=(_GRID,),
        in_specs=[
            pl.BlockSpec((NC, BR, H), lambda i: (0, i, 0)),
            pl.BlockSpec((BR, H), lambda i: (i, 0)),
            pl.BlockSpec((BR, 1), lambda i: (i, 0)),
            pl.BlockSpec((1, H), lambda i: (0, 0)),
            pl.BlockSpec((H, H), lambda i: (0, 0)),
        ],
        out_specs=[
            pl.BlockSpec((BR, H), lambda i: (i, 0)),
            pl.BlockSpec((BR, H), lambda i: (i, 0)),
        ],
        out_shape=[
            jax.ShapeDtypeStruct((N, H), jnp.float32),
            jax.ShapeDtypeStruct((N, H), jnp.float32),
        ],
    )(agg, hs, dinv, b, wnt)


def _tc_final(agg, hs, dinv, b, out1, out2, wlt, bl):
    def body(agg_ref, hs_ref, dinv_ref, b_ref, o1_ref, o2_ref, wl_ref, bl_ref,
             log_ref):
        y3 = _layer_out(agg_ref, hs_ref, dinv_ref, b_ref)
        acc = _dot(o1_ref[...], wl_ref[0:H, :])
        acc += _dot(o2_ref[...], wl_ref[H:2 * H, :])
        acc += _dot(y3, wl_ref[2 * H:3 * H, :])
        log_ref[...] = acc + bl_ref[...]

    return pl.pallas_call(
        body,
        grid=(_GRID,),
        in_specs=[
            pl.BlockSpec((NC, BR, H), lambda i: (0, i, 0)),
            pl.BlockSpec((BR, H), lambda i: (i, 0)),
            pl.BlockSpec((BR, 1), lambda i: (i, 0)),
            pl.BlockSpec((1, H), lambda i: (0, 0)),
            pl.BlockSpec((BR, H), lambda i: (i, 0)),
            pl.BlockSpec((BR, H), lambda i: (i, 0)),
            pl.BlockSpec((3 * H, C), lambda i: (0, 0)),
            pl.BlockSpec((1, C), lambda i: (0, 0)),
        ],
        out_specs=pl.BlockSpec((BR, C), lambda i: (i, 0)),
        out_shape=jax.ShapeDtypeStruct((N, C), jnp.float32),
    )(agg, hs, dinv, b, out1, out2, wlt, bl)


def kernel(x, edge_index, edge_weights, W1, b1, W2, b2, W3, b3, Wl, bl):
    src = edge_index[0]
    dst = edge_index[1]
    zeros_n = jnp.zeros((N,), jnp.float32)
    zeros_nh = jnp.zeros((N, H), jnp.float32)

    degs = _sc_deg(dst, edge_weights, zeros_n)
    dinv, hs1 = _tc_pre(x, W1.T, degs)
    agg1 = _sc_agg(hs1, src, dst, edge_weights, zeros_nh)
    out1, hs2 = _tc_post(agg1, hs1, dinv, b1[None, :], W2.T)
    agg2 = _sc_agg(hs2, src, dst, edge_weights, zeros_nh)
    out2, hs3 = _tc_post(agg2, hs2, dinv, b2[None, :], W3.T)
    agg3 = _sc_agg(hs3, src, dst, edge_weights, zeros_nh)
    return _tc_final(agg3, hs3, dinv, b3[None, :], out1, out2, Wl.T,
                     bl[None, :])


# speedup vs baseline: 15.5117x; 1.8533x over previous
import functools

import jax
import jax.numpy as jnp
from jax import lax
from jax.experimental import pallas as pl
from jax.experimental.pallas import tpu as pltpu
from jax.experimental.pallas import tpu_sc as plsc

N = 10000
E = 320000
H = 128
C = 16

NC = 2
NS = 16
NW = NC * NS
EPT = E // NW
CH = 80
NCHUNK = EPT // CH
WB_CH = 624
WB_LAST = N - (NS - 1) * WB_CH

_MESH = plsc.VectorSubcoreMesh(core_axis_name="c", subcore_axis_name="s")
_SC_PARAMS = pltpu.CompilerParams(needs_layout_passes=False)


@functools.partial(
    pl.kernel,
    mesh=_MESH,
    out_type=jax.ShapeDtypeStruct((NC, N), jnp.float32),
    compiler_params=_SC_PARAMS,
    scratch_types=[
        pltpu.VMEM((CH,), jnp.int32),
        pltpu.VMEM((CH,), jnp.float32),
        pltpu.VMEM_SHARED((N,), jnp.float32),
    ],
)
def _sc_deg(dst_hbm, w_hbm, zero_hbm, out_hbm, didx, wv, acc):
    c = lax.axis_index("c")
    s = lax.axis_index("s")
    wid = s * NC + c
    base = wid * EPT

    @pl.when(s == 0)
    def _():
        pltpu.sync_copy(zero_hbm, acc)

    plsc.subcore_barrier()

    def body(j, carry):
        off = pl.multiple_of(base + j * CH, 8)
        pltpu.sync_copy(dst_hbm.at[pl.ds(off, CH)], didx)
        pltpu.sync_copy(w_hbm.at[pl.ds(off, CH)], wv)
        pltpu.sync_copy(wv, acc.at[didx], add=True)
        return carry

    lax.fori_loop(0, NCHUNK, body, 0)
    plsc.subcore_barrier()

    @pl.when(s == 0)
    def _():
        pltpu.sync_copy(acc, out_hbm.at[c])


@functools.partial(
    pl.kernel,
    mesh=_MESH,
    out_type=jax.ShapeDtypeStruct((NC, N, H), jnp.float32),
    compiler_params=_SC_PARAMS,
    scratch_types=[
        pltpu.VMEM((CH,), jnp.int32),
        pltpu.VMEM((CH,), jnp.int32),
        pltpu.VMEM((CH,), jnp.int32),
        pltpu.VMEM((CH,), jnp.int32),
        pltpu.VMEM((CH,), jnp.float32),
        pltpu.VMEM((CH,), jnp.float32),
        pltpu.VMEM((CH, H), jnp.float32),
        pltpu.VMEM((CH, H), jnp.float32),
        pltpu.SemaphoreType.DMA,
        pltpu.SemaphoreType.DMA,
        pltpu.SemaphoreType.DMA,
        pltpu.SemaphoreType.DMA,
        pltpu.VMEM_SHARED((N, H), jnp.float32),
    ],
)
def _sc_agg(hs_hbm, src_hbm, dst_hbm, w_hbm, zero_hbm, out_hbm,
            sidx0, sidx1, didx0, didx1, wv0, wv1, rows0, rows1,
            semi0, semi1, semg0, semg1, acc):
    c = lax.axis_index("c")
    s = lax.axis_index("s")
    wid = s * NC + c
    base = wid * EPT
    sidx = (sidx0, sidx1)
    didx = (didx0, didx1)
    wv = (wv0, wv1)
    rows = (rows0, rows1)
    semi = (semi0, semi1)
    semg = (semg0, semg1)

    @pl.when(s == 0)
    def _():
        pltpu.sync_copy(zero_hbm, acc)

    plsc.subcore_barrier()

    def idx_start(ck, b):
        off = pl.multiple_of(base + ck * CH, 8)
        pltpu.async_copy(src_hbm.at[pl.ds(off, CH)], sidx[b], semi[b])
        pltpu.async_copy(dst_hbm.at[pl.ds(off, CH)], didx[b], semi[b])
        pltpu.async_copy(w_hbm.at[pl.ds(off, CH)], wv[b], semi[b])

    def idx_wait(b):
        sl = pl.ds(0, CH)
        pltpu.make_async_copy(src_hbm.at[sl], sidx[b], semi[b]).wait()
        pltpu.make_async_copy(dst_hbm.at[sl], didx[b], semi[b]).wait()
        pltpu.make_async_copy(w_hbm.at[sl], wv[b], semi[b]).wait()

    def gather_start(b):
        pltpu.async_copy(hs_hbm.at[sidx[b]], rows[b], semg[b])

    def gather_wait(b):
        pltpu.make_async_copy(hs_hbm.at[sidx[b]], rows[b], semg[b]).wait()

    def compute(b):
        for g in range(CH // 16):
            wvec = wv[b][pl.ds(g * 16, 16)]
            for l in range(16):
                r = g * 16 + l
                wb = wvec.at[jnp.full((16,), l, jnp.int32)].get(
                    mode="promise_in_bounds")
                for col in range(H // 16):
                    sl = pl.ds(col * 16, 16)
                    rows[b][r, sl] = rows[b][r, sl] * wb

    def scatter(b):
        pltpu.sync_copy(rows[b], acc.at[didx[b]], add=True)

    idx_start(0, 0)
    idx_start(1, 1)
    idx_wait(0)
    gather_start(0)

    def body(j, carry):
        for k in range(2):
            ck = 2 * j + k
            b = k
            b2 = 1 - k
            gather_wait(b)
            idx_wait(b2)
            gather_start(b2)
            compute(b)
            scatter(b)

            @pl.when(ck + 2 < NCHUNK)
            def _():
                idx_start(ck + 2, b)
        return carry

    lax.fori_loop(0, (NCHUNK - 1) // 2, body, 0)
    lb = (NCHUNK - 1) % 2
    gather_wait(lb)
    compute(lb)
    scatter(lb)

    plsc.subcore_barrier()

    @pl.when(s < NS - 1)
    def _():
        rsl = pl.ds(pl.multiple_of(s * WB_CH, 8), WB_CH)
        pltpu.sync_copy(acc.at[rsl], out_hbm.at[c].at[rsl])

    @pl.when(s == NS - 1)
    def _():
        rsl = pl.ds((NS - 1) * WB_CH, WB_LAST)
        pltpu.sync_copy(acc.at[rsl], out_hbm.at[c].at[rsl])


BR = 2000
_GRID = N // BR


def _dot(a, b):
    return jnp.dot(a, b, preferred_element_type=jnp.float32,
                   precision=lax.Precision.HIGHEST)


def _tc_pre(x, w1t, degs):
    def body(x_ref, w_ref, deg_ref, dinv_ref, hs_ref):
        deg = deg_ref[:, 0] + deg_ref[:, 1] + 1.0
        dinv = jnp.where(deg > 0, lax.rsqrt(deg), 0.0)
        dinv_ref[...] = dinv[:, None]
        hs_ref[...] = _dot(x_ref[...], w_ref[...]) * dinv[:, None]

    return pl.pallas_call(
        body,
        grid=(_GRID,),
        in_specs=[
            pl.BlockSpec((BR, H), lambda i: (i, 0)),
            pl.BlockSpec((H, H), lambda i: (0, 0)),
            pl.BlockSpec((BR, NC), lambda i: (i, 0)),
        ],
        out_specs=[
            pl.BlockSpec((BR, 1), lambda i: (i, 0)),
            pl.BlockSpec((BR, H), lambda i: (i, 0)),
        ],
        out_shape=[
            jax.ShapeDtypeStruct((N, 1), jnp.float32),
            jax.ShapeDtypeStruct((N, H), jnp.float32),
        ],
    )(x, w1t, degs.T)


def _layer_out(agg_ref, hs_ref, dinv_ref, b_ref):
    y = (agg_ref[0] + agg_ref[1] + hs_ref[...]) * dinv_ref[...] + b_ref[...]
    nrm = jnp.sqrt(jnp.sum(y * y, axis=1, keepdims=True))
    y = y / jnp.maximum(nrm, 1e-12)
    return jnp.maximum(y, 0.0)


def _tc_post(agg, hs, dinv, b, wnt):
    def body(agg_ref, hs_ref, dinv_ref, b_ref, w_ref, out_ref, hsn_ref):
        y = _layer_out(agg_ref, hs_ref, dinv_ref, b_ref)
        out_ref[...] = y
        hsn_ref[...] = _dot(y, w_ref[...]) * dinv_ref[...]

    return pl.pallas_call(
        body,
        grid=(_GRID,),
        in_specs=[
            pl.BlockSpec((NC, BR, H), lambda i: (0, i, 0)),
            pl.BlockSpec((BR, H), lambda i: (i, 0)),
            pl.BlockSpec((BR, 1), lambda i: (i, 0)),
            pl.BlockSpec((1, H), lambda i: (0, 0)),
            pl.BlockSpec((H, H), lambda i: (0, 0)),
        ],
        out_specs=[
            pl.BlockSpec((BR, H), lambda i: (i, 0)),
            pl.BlockSpec((BR, H), lambda i: (i, 0)),
        ],
        out_shape=[
            jax.ShapeDtypeStruct((N, H), jnp.float32),
            jax.ShapeDtypeStruct((N, H), jnp.float32),
        ],
    )(agg, hs, dinv, b, wnt)


def _tc_final(agg, hs, dinv, b, out1, out2, wlt, bl):
    def body(agg_ref, hs_ref, dinv_ref, b_ref, o1_ref, o2_ref, wl_ref, bl_ref,
             log_ref):
        y3 = _layer_out(agg_ref, hs_ref, dinv_ref, b_ref)
        acc = _dot(o1_ref[...], wl_ref[0:H, :])
        acc += _dot(o2_ref[...], wl_ref[H:2 * H, :])
        acc += _dot(y3, wl_ref[2 * H:3 * H, :])
        log_ref[...] = acc + bl_ref[...]

    return pl.pallas_call(
        body,
        grid=(_GRID,),
        in_specs=[
            pl.BlockSpec((NC, BR, H), lambda i: (0, i, 0)),
            pl.BlockSpec((BR, H), lambda i: (i, 0)),
            pl.BlockSpec((BR, 1), lambda i: (i, 0)),
            pl.BlockSpec((1, H), lambda i: (0, 0)),
            pl.BlockSpec((BR, H), lambda i: (i, 0)),
            pl.BlockSpec((BR, H), lambda i: (i, 0)),
            pl.BlockSpec((3 * H, C), lambda i: (0, 0)),
            pl.BlockSpec((1, C), lambda i: (0, 0)),
        ],
        out_specs=pl.BlockSpec((BR, C), lambda i: (i, 0)),
        out_shape=jax.ShapeDtypeStruct((N, C), jnp.float32),
    )(agg, hs, dinv, b, out1, out2, wlt, bl)


def kernel(x, edge_index, edge_weights, W1, b1, W2, b2, W3, b3, Wl, bl):
    src = edge_index[0]
    dst = edge_index[1]
    zeros_n = jnp.zeros((N,), jnp.float32)
    zeros_nh = jnp.zeros((N, H), jnp.float32)

    degs = _sc_deg(dst, edge_weights, zeros_n)
    dinv, hs1 = _tc_pre(x, W1.T, degs)
    agg1 = _sc_agg(hs1, src, dst, edge_weights, zeros_nh)
    out1, hs2 = _tc_post(agg1, hs1, dinv, b1[None, :], W2.T)
    agg2 = _sc_agg(hs2, src, dst, edge_weights, zeros_nh)
    out2, hs3 = _tc_post(agg2, hs2, dinv, b2[None, :], W3.T)
    agg3 = _sc_agg(hs3, src, dst, edge_weights, zeros_nh)
    return _tc_final(agg3, hs3, dinv, b3[None, :], out1, out2, Wl.T,
                     bl[None, :])
